# Initial kernel scaffold; baseline (speedup 1.0000x reference)
import jax
import jax.numpy as jnp
from jax.experimental import pallas as pl


def kernel(source_nodes, destination_nodes, negative_nodes, edge_times, edge_idxs, n_neighbors, node_emb, edge_feat, nbr_nodes, nbr_eidx, nbr_times, time_w, time_b, Wq, Wk, Wv, W1, b1, W2, b2, res_scale):
    raise NotImplementedError("write your pallas kernel here")


# speedup vs baseline: 3.0993x; 3.0993x over previous
import functools
import math

import jax
import jax.numpy as jnp
from jax import lax
from jax.experimental import pallas as pl
from jax.experimental.pallas import tpu as pltpu
from jax.experimental.pallas import tpu_sc as plsc

D = 256
DE = 16
K = 20
H = 2
DH = D // H
TBLW = 64


def _sc_dims():
    try:
        info = plsc.get_sparse_core_info()
        return int(info.num_cores), int(info.num_subcores)
    except Exception:
        return 2, 16


def _sc_gather_queries(nodes, tbl, node_emb):
    B3 = nodes.shape[0]
    NC, NS = _sc_dims()
    NW = NC * NS
    assert B3 % NW == 0
    per = B3 // NW
    mesh = plsc.VectorSubcoreMesh(
        core_axis_name="c", subcore_axis_name="s", num_cores=NC, num_subcores=NS)

    @functools.partial(
        pl.kernel,
        out_type=[
            jax.ShapeDtypeStruct((B3, TBLW), jnp.int32),
            jax.ShapeDtypeStruct((B3, D), jnp.float32),
        ],
        mesh=mesh,
        compiler_params=pltpu.CompilerParams(use_tc_tiling_on_sc=False),
        scratch_types=[
            pltpu.VMEM((per,), jnp.int32),
            pltpu.VMEM((per, TBLW), jnp.int32),
            pltpu.VMEM((per, D), jnp.float32),
            pltpu.SemaphoreType.DMA,
            pltpu.SemaphoreType.DMA,
        ],
    )
    def body(nodes_hbm, tbl_hbm, emb_hbm, g_out, x_out, idx_v, tbl_v, x_v, s1, s2):
        wid = lax.axis_index("s") * NC + lax.axis_index("c")
        base = wid * per
        pltpu.sync_copy(nodes_hbm.at[pl.ds(base, per)], idx_v)
        c1 = pltpu.async_copy(tbl_hbm.at[idx_v], tbl_v, s1)
        c2 = pltpu.async_copy(emb_hbm.at[idx_v], x_v, s2)
        c1.wait()
        c2.wait()
        pltpu.sync_copy(tbl_v, g_out.at[pl.ds(base, per)])
        pltpu.sync_copy(x_v, x_out.at[pl.ds(base, per)])

    return body(nodes, tbl, node_emb)


def _sc_gather_neighbors(nidx2d, eidx2d, node_emb, edge_feat, M):
    NC, NS = _sc_dims()
    NW = NC * NS
    CH = 128
    assert M % (NW * CH) == 0
    n_ch = M // (NW * CH)
    per = n_ch * CH
    mesh = plsc.VectorSubcoreMesh(
        core_axis_name="c", subcore_axis_name="s", num_cores=NC, num_subcores=NS)

    @functools.partial(
        pl.kernel,
        out_type=[
            jax.ShapeDtypeStruct((M, D), jnp.float32),
            jax.ShapeDtypeStruct((M, DE), jnp.float32),
        ],
        mesh=mesh,
        compiler_params=pltpu.CompilerParams(use_tc_tiling_on_sc=False),
        scratch_types=[
            pltpu.VMEM((n_ch, CH), jnp.int32),
            pltpu.VMEM((n_ch, CH), jnp.int32),
            pltpu.VMEM((2, CH, D), jnp.float32),
            pltpu.VMEM((2, CH, DE), jnp.float32),
            pltpu.SemaphoreType.DMA,
            pltpu.SemaphoreType.DMA,
            pltpu.SemaphoreType.DMA,
            pltpu.SemaphoreType.DMA,
        ],
    )
    def body(ni_hbm, ei_hbm, emb_hbm, ef_hbm, nf_out, ef_out,
             ni_v, ei_v, nbuf, ebuf, sn0, sn1, se0, se1):
        wid = lax.axis_index("s") * NC + lax.axis_index("c")
        base = wid * per
        pltpu.sync_copy(ni_hbm.at[pl.ds(wid * n_ch, n_ch)], ni_v)
        pltpu.sync_copy(ei_hbm.at[pl.ds(wid * n_ch, n_ch)], ei_v)
        sn = (sn0, sn1)
        se = (se0, se1)

        def writeback(j, b):
            off = base + j * CH
            pltpu.sync_copy(nbuf.at[b], nf_out.at[pl.ds(off, CH)])
            pltpu.sync_copy(ebuf.at[b], ef_out.at[pl.ds(off, CH)])

        prev = None
        for j in range(n_ch):
            b = j % 2
            cn = pltpu.async_copy(emb_hbm.at[ni_v.at[j]], nbuf.at[b], sn[b])
            ce = pltpu.async_copy(ef_hbm.at[ei_v.at[j]], ebuf.at[b], se[b])
            if prev is not None:
                pcn, pce, pj, pb = prev
                pcn.wait()
                pce.wait()
                writeback(pj, pb)
            prev = (cn, ce, j, b)
        pcn, pce, pj, pb = prev
        pcn.wait()
        pce.wait()
        writeback(pj, pb)

    return body(nidx2d, eidx2d, node_emb, edge_feat)


def _tc_body(nn_ref, rs_ref, ts_ref, nts_ref, x_ref, nf_ref, ef_ref,
             tw_ref, tb_ref, wqx_ref, wqt_ref, wkn_ref, wke_ref, wkt_ref,
             wvn_ref, wve_ref, wvt_ref, w1o_ref, w1x_ref, b1_ref, w2_ref,
             b2_ref, out_ref):
    R = ts_ref.shape[0]
    f32 = jnp.float32
    dot = functools.partial(jnp.dot, preferred_element_type=f32)
    x = x_ref[...]
    nf = nf_ref[...]
    ef = ef_ref[...]
    tw = tw_ref[...]
    tb = tb_ref[...]
    delta = ts_ref[...] - nts_ref[...]
    te = jnp.cos(delta[:, :, None] * tw.reshape(1, 1, D) + tb.reshape(1, 1, D))
    te = te.reshape(R * K, D)
    kk = dot(nf, wkn_ref[...]) + dot(ef, wke_ref[...]) + dot(te, wkt_ref[...])
    vv = dot(nf, wvn_ref[...]) + dot(ef, wve_ref[...]) + dot(te, wvt_ref[...])
    q = dot(x, wqx_ref[...]) + dot(jnp.cos(tb), wqt_ref[...])
    k3 = kk.reshape(R, K, D)
    v3 = vv.reshape(R, K, D)
    nn = nn_ref[0, 0]
    kiota = lax.broadcasted_iota(jnp.int32, (R, K), 1)
    inv_sqrt = f32(1.0 / math.sqrt(DH))
    outs = []
    for h in range(H):
        sl = slice(h * DH, (h + 1) * DH)
        q_h = q[:, sl]
        k_h = k3[:, :, sl]
        v_h = v3[:, :, sl]
        scores = jnp.sum(q_h[:, None, :] * k_h, axis=-1) * inv_sqrt
        scores = jnp.where(kiota < nn, scores, -jnp.inf)
        m = jnp.max(scores, axis=1, keepdims=True)
        e = jnp.exp(scores - m)
        attn = e / jnp.sum(e, axis=1, keepdims=True)
        outs.append(jnp.sum(attn[:, :, None] * v_h, axis=1))
    out = jnp.concatenate(outs, axis=-1)
    hh = dot(out, w1o_ref[...]) + dot(x, w1x_ref[...]) + b1_ref[...]
    hh = jnp.maximum(hh, 0.0)
    out_ref[...] = dot(hh, w2_ref[...]) + b2_ref[...] + rs_ref[0, 0] * x


def _tc_embed(R, B3, interpret=False):
    G = B3 // R
    row = lambda i: (i, 0)
    fix = lambda i: (0, 0)
    smem = pl.BlockSpec(memory_space=pltpu.SMEM)
    return pl.pallas_call(
        _tc_body,
        grid=(G,),
        in_specs=[
            smem,
            smem,
            pl.BlockSpec((R, 1), row),
            pl.BlockSpec((R, K), row),
            pl.BlockSpec((R, D), row),
            pl.BlockSpec((R * K, D), row),
            pl.BlockSpec((R * K, DE), row),
            pl.BlockSpec((1, D), fix),
            pl.BlockSpec((1, D), fix),
            pl.BlockSpec((D, D), fix),
            pl.BlockSpec((D, D), fix),
            pl.BlockSpec((D, D), fix),
            pl.BlockSpec((DE, D), fix),
            pl.BlockSpec((D, D), fix),
            pl.BlockSpec((D, D), fix),
            pl.BlockSpec((DE, D), fix),
            pl.BlockSpec((D, D), fix),
            pl.BlockSpec((D, D), fix),
            pl.BlockSpec((D, D), fix),
            pl.BlockSpec((1, D), fix),
            pl.BlockSpec((D, D), fix),
            pl.BlockSpec((1, D), fix),
        ],
        out_specs=pl.BlockSpec((R, D), row),
        out_shape=jax.ShapeDtypeStruct((B3, D), jnp.float32),
        interpret=interpret,
    )


def kernel(source_nodes, destination_nodes, negative_nodes, edge_times, edge_idxs,
           n_neighbors, node_emb, edge_feat, nbr_nodes, nbr_eidx, nbr_times,
           time_w, time_b, Wq, Wk, Wv, W1, b1, W2, b2, res_scale):
    i32 = jnp.int32
    f32 = jnp.float32
    nodes = jnp.concatenate([source_nodes, destination_nodes, negative_nodes]).astype(i32)
    ts3 = jnp.concatenate([edge_times, edge_times, edge_times]).astype(f32)
    B3 = nodes.shape[0]
    node_emb = node_emb.astype(f32)
    edge_feat = edge_feat.astype(f32)

    Nn = nbr_nodes.shape[0]
    tbl = jnp.concatenate([
        nbr_nodes.astype(i32),
        nbr_eidx.astype(i32),
        lax.bitcast_convert_type(nbr_times.astype(f32), i32),
        jnp.zeros((Nn, TBLW - 3 * K), i32),
    ], axis=1)

    g, x = _sc_gather_queries(nodes, tbl, node_emb)
    nbrs = g[:, :K].reshape(-1)
    eidx = g[:, K:2 * K].reshape(-1)
    nts = lax.bitcast_convert_type(g[:, 2 * K:3 * K], f32)
    M = B3 * K
    nf, ef = _sc_gather_neighbors(
        nbrs.reshape(M // 128, 128), eidx.reshape(M // 128, 128),
        node_emb, edge_feat, M)

    nn = jnp.asarray(n_neighbors, i32).reshape(1, 1)
    rs = jnp.asarray(res_scale, f32).reshape(1, 1)
    R = 128
    emb = _tc_embed(R, B3)(
        nn, rs, ts3.reshape(B3, 1), nts, x, nf, ef,
        time_w.astype(f32).reshape(1, D), time_b.astype(f32).reshape(1, D),
        Wq[:D], Wq[D:], Wk[:D], Wk[D:D + DE], Wk[D + DE:],
        Wv[:D], Wv[D:D + DE], Wv[D + DE:],
        W1[:D], W1[D:], b1.reshape(1, D), W2, b2.reshape(1, D))

    Bn = source_nodes.shape[0]
    return emb[:Bn], emb[Bn:2 * Bn], emb[2 * Bn:]


# speedup vs baseline: 5.7057x; 1.8410x over previous
import functools
import math

import jax
import jax.numpy as jnp
from jax import lax
from jax.experimental import pallas as pl
from jax.experimental.pallas import tpu as pltpu
from jax.experimental.pallas import tpu_sc as plsc

D = 256
DE = 16
K = 20
H = 2
DH = D // H
TBLW = 64

_COS_C1 = 6.28125
_COS_C2 = 0.0019340515136718750
_COS_C3 = 1.2554227678489685e-06
_INV_2PI = 0.15915494309189535
_COS_POLY = (0.9999994, -0.49999544, 0.041660894, -0.001386227,
             2.424664e-05, -2.2163067e-07)


def _fast_cos(t):
    f = jnp.float32
    n = jnp.floor(t * f(_INV_2PI) + f(0.5))
    r = ((t - n * f(_COS_C1)) - n * f(_COS_C2)) - n * f(_COS_C3)
    z = r * r
    acc = jnp.full_like(z, f(_COS_POLY[-1]))
    for c in _COS_POLY[-2::-1]:
        acc = acc * z + f(c)
    return acc


def _sc_dims():
    try:
        info = plsc.get_sparse_core_info()
        return int(info.num_cores), int(info.num_subcores)
    except Exception:
        return 2, 16


def _sc_gather_tables(nodes, tbl):
    B3 = nodes.shape[0]
    NC, NS = _sc_dims()
    NW = NC * NS
    assert B3 % NW == 0
    per = B3 // NW
    mesh = plsc.VectorSubcoreMesh(
        core_axis_name="c", subcore_axis_name="s", num_cores=NC, num_subcores=NS)

    @functools.partial(
        pl.kernel,
        out_type=jax.ShapeDtypeStruct((B3, TBLW), jnp.int32),
        mesh=mesh,
        compiler_params=pltpu.CompilerParams(use_tc_tiling_on_sc=False),
        scratch_types=[
            pltpu.VMEM((per,), jnp.int32),
            pltpu.VMEM((per, TBLW), jnp.int32),
            pltpu.SemaphoreType.DMA,
        ],
    )
    def body(nodes_hbm, tbl_hbm, g_out, idx_v, tbl_v, s1):
        wid = lax.axis_index("s") * NC + lax.axis_index("c")
        base = wid * per
        pltpu.sync_copy(nodes_hbm.at[pl.ds(base, per)], idx_v)
        pltpu.async_copy(tbl_hbm.at[idx_v], tbl_v, s1).wait()
        pltpu.sync_copy(tbl_v, g_out.at[pl.ds(base, per)])

    return body(nodes, tbl)


def _sc_gather_embeddings(nodes, nidx3d, node_emb, B3, M):
    NC, NS = _sc_dims()
    NW = NC * NS
    CH = 128
    assert B3 % NW == 0 and M % (NW * CH) == 0
    per_q = B3 // NW
    n_ch = M // (NW * CH)
    per = n_ch * CH
    mesh = plsc.VectorSubcoreMesh(
        core_axis_name="c", subcore_axis_name="s", num_cores=NC, num_subcores=NS)

    @functools.partial(
        pl.kernel,
        out_type=[
            jax.ShapeDtypeStruct((B3, D), jnp.float32),
            jax.ShapeDtypeStruct((M, D), jnp.float32),
        ],
        mesh=mesh,
        scratch_types=[
            pltpu.VMEM((per_q,), jnp.int32),
            pltpu.VMEM((n_ch, CH), jnp.int32),
            pltpu.VMEM((per_q, D), jnp.float32),
            pltpu.VMEM((2, CH, D), jnp.float32),
            pltpu.SemaphoreType.DMA,
            pltpu.SemaphoreType.DMA,
            pltpu.SemaphoreType.DMA,
        ],
    )
    def body(nodes_hbm, ni_hbm, emb_hbm, x_out, nf_out,
             qidx_v, ni_v, x_v, nbuf, sq, sn0, sn1):
        wid = lax.axis_index("s") * NC + lax.axis_index("c")
        qbase = wid * per_q
        base = wid * per
        pltpu.sync_copy(nodes_hbm.at[pl.ds(qbase, per_q)], qidx_v)
        pltpu.sync_copy(ni_hbm.at[wid], ni_v)
        cq = pltpu.async_copy(emb_hbm.at[qidx_v], x_v, sq)
        sn = (sn0, sn1)
        prev = None
        for j in range(n_ch):
            b = j % 2
            cn = pltpu.async_copy(emb_hbm.at[ni_v.at[j]], nbuf.at[b], sn[b])
            if prev is not None:
                pcn, pj, pb = prev
                pcn.wait()
                pltpu.sync_copy(nbuf.at[pb], nf_out.at[pl.ds(base + pj * CH, CH)])
            prev = (cn, j, b)
        cq.wait()
        pltpu.sync_copy(x_v, x_out.at[pl.ds(qbase, per_q)])
        pcn, pj, pb = prev
        pcn.wait()
        pltpu.sync_copy(nbuf.at[pb], nf_out.at[pl.ds(base + pj * CH, CH)])

    return body(nodes, nidx3d, node_emb)


def _sc_gather_edges(eidx2d, edge_feat, M):
    NC, NS = _sc_dims()
    NW = NC * NS
    CH = 128
    assert M % (NW * CH) == 0
    n_ch = M // (NW * CH)
    per = n_ch * CH
    mesh = plsc.VectorSubcoreMesh(
        core_axis_name="c", subcore_axis_name="s", num_cores=NC, num_subcores=NS)

    @functools.partial(
        pl.kernel,
        out_type=jax.ShapeDtypeStruct((M, DE), jnp.float32),
        mesh=mesh,
        compiler_params=pltpu.CompilerParams(use_tc_tiling_on_sc=False),
        scratch_types=[
            pltpu.VMEM((n_ch, CH), jnp.int32),
            pltpu.VMEM((2, CH, DE), jnp.float32),
            pltpu.SemaphoreType.DMA,
            pltpu.SemaphoreType.DMA,
        ],
    )
    def body(ei_hbm, ef_hbm, ef_out, ei_v, ebuf, se0, se1):
        wid = lax.axis_index("s") * NC + lax.axis_index("c")
        base = wid * per
        pltpu.sync_copy(ei_hbm.at[pl.ds(wid * n_ch, n_ch)], ei_v)
        se = (se0, se1)
        prev = None
        for j in range(n_ch):
            b = j % 2
            ce = pltpu.async_copy(ef_hbm.at[ei_v.at[j]], ebuf.at[b], se[b])
            if prev is not None:
                pce, pj, pb = prev
                pce.wait()
                pltpu.sync_copy(ebuf.at[pb], ef_out.at[pl.ds(base + pj * CH, CH)])
            prev = (ce, j, b)
        pce, pj, pb = prev
        pce.wait()
        pltpu.sync_copy(ebuf.at[pb], ef_out.at[pl.ds(base + pj * CH, CH)])

    return body(eidx2d, edge_feat)


def _tc_body(nn_ref, rs_ref, ts_ref, nts_ref, x_ref, nf_ref, ef_ref,
             tw_ref, tb_ref, wqx_ref, wqt_ref, wkn_ref, wke_ref, wkt_ref,
             wvn_ref, wve_ref, wvt_ref, w1o_ref, w1x_ref, b1_ref, w2_ref,
             b2_ref, out_ref):
    R = ts_ref.shape[0]
    f32 = jnp.float32
    dot = functools.partial(jnp.dot, preferred_element_type=f32)
    x = x_ref[...]
    nf = nf_ref[...]
    ef = ef_ref[...]
    tw = tw_ref[...]
    tb = tb_ref[...]
    delta = ts_ref[...] - nts_ref[...]
    te = _fast_cos(delta[:, :, None] * tw.reshape(1, 1, D) + tb.reshape(1, 1, D))
    te = te.reshape(R * K, D)
    kk = dot(nf, wkn_ref[...]) + dot(ef, wke_ref[...]) + dot(te, wkt_ref[...])
    vv = dot(nf, wvn_ref[...]) + dot(ef, wve_ref[...]) + dot(te, wvt_ref[...])
    q = dot(x, wqx_ref[...]) + dot(jnp.cos(tb), wqt_ref[...])
    k3 = kk.reshape(R, K, D)
    v3 = vv.reshape(R, K, D)
    nn = nn_ref[0, 0]
    kiota = lax.broadcasted_iota(jnp.int32, (R, K), 1)
    inv_sqrt = f32(1.0 / math.sqrt(DH))
    outs = []
    for h in range(H):
        sl = slice(h * DH, (h + 1) * DH)
        q_h = q[:, sl]
        k_h = k3[:, :, sl]
        v_h = v3[:, :, sl]
        scores = jnp.sum(q_h[:, None, :] * k_h, axis=-1) * inv_sqrt
        scores = jnp.where(kiota < nn, scores, -jnp.inf)
        m = jnp.max(scores, axis=1, keepdims=True)
        e = jnp.exp(scores - m)
        attn = e / jnp.sum(e, axis=1, keepdims=True)
        outs.append(jnp.sum(attn[:, :, None] * v_h, axis=1))
    out = jnp.concatenate(outs, axis=-1)
    hh = dot(out, w1o_ref[...]) + dot(x, w1x_ref[...]) + b1_ref[...]
    hh = jnp.maximum(hh, 0.0)
    out_ref[...] = dot(hh, w2_ref[...]) + b2_ref[...] + rs_ref[0, 0] * x


def _tc_embed(R, B3, interpret=False):
    G = B3 // R
    row = lambda i: (i, 0)
    fix = lambda i: (0, 0)
    smem = pl.BlockSpec(memory_space=pltpu.SMEM)
    return pl.pallas_call(
        _tc_body,
        grid=(G,),
        in_specs=[
            smem,
            smem,
            pl.BlockSpec((R, 1), row),
            pl.BlockSpec((R, K), row),
            pl.BlockSpec((R, D), row),
            pl.BlockSpec((R * K, D), row),
            pl.BlockSpec((R * K, DE), row),
            pl.BlockSpec((1, D), fix),
            pl.BlockSpec((1, D), fix),
            pl.BlockSpec((D, D), fix),
            pl.BlockSpec((D, D), fix),
            pl.BlockSpec((D, D), fix),
            pl.BlockSpec((DE, D), fix),
            pl.BlockSpec((D, D), fix),
            pl.BlockSpec((D, D), fix),
            pl.BlockSpec((DE, D), fix),
            pl.BlockSpec((D, D), fix),
            pl.BlockSpec((D, D), fix),
            pl.BlockSpec((D, D), fix),
            pl.BlockSpec((1, D), fix),
            pl.BlockSpec((D, D), fix),
            pl.BlockSpec((1, D), fix),
        ],
        out_specs=pl.BlockSpec((R, D), row),
        out_shape=jax.ShapeDtypeStruct((B3, D), jnp.float32),
        interpret=interpret,
    )


def kernel(source_nodes, destination_nodes, negative_nodes, edge_times, edge_idxs,
           n_neighbors, node_emb, edge_feat, nbr_nodes, nbr_eidx, nbr_times,
           time_w, time_b, Wq, Wk, Wv, W1, b1, W2, b2, res_scale):
    i32 = jnp.int32
    f32 = jnp.float32
    nodes = jnp.concatenate([source_nodes, destination_nodes, negative_nodes]).astype(i32)
    ts3 = jnp.concatenate([edge_times, edge_times, edge_times]).astype(f32)
    B3 = nodes.shape[0]
    node_emb = node_emb.astype(f32)
    edge_feat = edge_feat.astype(f32)

    Nn = nbr_nodes.shape[0]
    tbl = jnp.concatenate([
        nbr_nodes.astype(i32),
        nbr_eidx.astype(i32),
        lax.bitcast_convert_type(nbr_times.astype(f32), i32),
        jnp.zeros((Nn, TBLW - 3 * K), i32),
    ], axis=1)

    g = _sc_gather_tables(nodes, tbl)
    NC, NS = _sc_dims()
    NW = NC * NS
    M = B3 * K
    nbrs3d = g[:, :K].reshape(NW, M // (NW * 128), 128)
    eidx2d = g[:, K:2 * K].reshape(M // 128, 128)
    nts = lax.bitcast_convert_type(g[:, 2 * K:3 * K], f32)

    x, nf = _sc_gather_embeddings(nodes, nbrs3d, node_emb, B3, M)
    ef = _sc_gather_edges(eidx2d, edge_feat, M)

    nn = jnp.asarray(n_neighbors, i32).reshape(1, 1)
    rs = jnp.asarray(res_scale, f32).reshape(1, 1)
    R = 256
    emb = _tc_embed(R, B3)(
        nn, rs, ts3.reshape(B3, 1), nts, x, nf, ef,
        time_w.astype(f32).reshape(1, D), time_b.astype(f32).reshape(1, D),
        Wq[:D], Wq[D:], Wk[:D], Wk[D:D + DE], Wk[D + DE:],
        Wv[:D], Wv[D:D + DE], Wv[D + DE:],
        W1[:D], W1[D:], b1.reshape(1, D), W2, b2.reshape(1, D))

    Bn = source_nodes.shape[0]
    return emb[:Bn], emb[Bn:2 * Bn], emb[2 * Bn:]


# speedup vs baseline: 5.9450x; 1.0419x over previous
import functools
import math

import jax
import jax.numpy as jnp
from jax import lax
from jax.experimental import pallas as pl
from jax.experimental.pallas import tpu as pltpu
from jax.experimental.pallas import tpu_sc as plsc

D = 256
DE = 16
K = 20
H = 2
DH = D // H
TBLW = 64

_COS_C1 = 6.28125
_COS_C2 = 0.0019340515136718750
_COS_C3 = 1.2554227678489685e-06
_INV_2PI = 0.15915494309189535
_COS_POLY = (0.9999994, -0.49999544, 0.041660894, -0.001386227,
             2.424664e-05, -2.2163067e-07)


def _fast_cos(t):
    f = jnp.float32
    n = jnp.floor(t * f(_INV_2PI) + f(0.5))
    r = ((t - n * f(_COS_C1)) - n * f(_COS_C2)) - n * f(_COS_C3)
    z = r * r
    acc = jnp.full_like(z, f(_COS_POLY[-1]))
    for c in _COS_POLY[-2::-1]:
        acc = acc * z + f(c)
    return acc


def _sc_dims():
    try:
        info = plsc.get_sparse_core_info()
        return int(info.num_cores), int(info.num_subcores)
    except Exception:
        return 2, 16


def _sc_gather_tables(nodes, tbl):
    B3 = nodes.shape[0]
    NC, NS = _sc_dims()
    NW = NC * NS
    assert B3 % NW == 0
    per = B3 // NW
    mesh = plsc.VectorSubcoreMesh(
        core_axis_name="c", subcore_axis_name="s", num_cores=NC, num_subcores=NS)

    @functools.partial(
        pl.kernel,
        out_type=jax.ShapeDtypeStruct((B3, TBLW), jnp.int32),
        mesh=mesh,
        compiler_params=pltpu.CompilerParams(use_tc_tiling_on_sc=False),
        scratch_types=[
            pltpu.VMEM((per,), jnp.int32),
            pltpu.VMEM((per, TBLW), jnp.int32),
            pltpu.SemaphoreType.DMA,
        ],
    )
    def body(nodes_hbm, tbl_hbm, g_out, idx_v, tbl_v, s1):
        wid = lax.axis_index("s") * NC + lax.axis_index("c")
        base = wid * per
        pltpu.sync_copy(nodes_hbm.at[pl.ds(base, per)], idx_v)
        pltpu.async_copy(tbl_hbm.at[idx_v], tbl_v, s1).wait()
        pltpu.sync_copy(tbl_v, g_out.at[pl.ds(base, per)])

    return body(nodes, tbl)


def _sc_gather_embeddings(nodes, nidx3d, node_emb, B3, M):
    NC, NS = _sc_dims()
    NW = NC * NS
    CH = 128
    assert B3 % NW == 0 and M % (NW * CH) == 0
    per_q = B3 // NW
    n_ch = M // (NW * CH)
    per = n_ch * CH
    mesh = plsc.VectorSubcoreMesh(
        core_axis_name="c", subcore_axis_name="s", num_cores=NC, num_subcores=NS)

    @functools.partial(
        pl.kernel,
        out_type=[
            jax.ShapeDtypeStruct((B3, D), jnp.float32),
            jax.ShapeDtypeStruct((M, D), jnp.float32),
        ],
        mesh=mesh,
        scratch_types=[
            pltpu.VMEM((per_q,), jnp.int32),
            pltpu.VMEM((n_ch, CH), jnp.int32),
            pltpu.VMEM((per_q, D), jnp.float32),
            pltpu.VMEM((2, CH, D), jnp.float32),
            pltpu.SemaphoreType.DMA,
            pltpu.SemaphoreType.DMA,
            pltpu.SemaphoreType.DMA,
        ],
    )
    def body(nodes_hbm, ni_hbm, emb_hbm, x_out, nf_out,
             qidx_v, ni_v, x_v, nbuf, sq, sn0, sn1):
        wid = lax.axis_index("s") * NC + lax.axis_index("c")
        qbase = wid * per_q
        base = wid * per
        pltpu.sync_copy(nodes_hbm.at[pl.ds(qbase, per_q)], qidx_v)
        pltpu.sync_copy(ni_hbm.at[wid], ni_v)
        cq = pltpu.async_copy(emb_hbm.at[qidx_v], x_v, sq)
        sn = (sn0, sn1)
        prev = None
        for j in range(n_ch):
            b = j % 2
            cn = pltpu.async_copy(emb_hbm.at[ni_v.at[j]], nbuf.at[b], sn[b])
            if prev is not None:
                pcn, pj, pb = prev
                pcn.wait()
                pltpu.sync_copy(nbuf.at[pb], nf_out.at[pl.ds(base + pj * CH, CH)])
            prev = (cn, j, b)
        cq.wait()
        pltpu.sync_copy(x_v, x_out.at[pl.ds(qbase, per_q)])
        pcn, pj, pb = prev
        pcn.wait()
        pltpu.sync_copy(nbuf.at[pb], nf_out.at[pl.ds(base + pj * CH, CH)])

    return body(nodes, nidx3d, node_emb)


def _sc_gather_edges(eidx2d, edge_feat, M):
    NC, NS = _sc_dims()
    NW = NC * NS
    CH = 128
    assert M % (NW * CH) == 0
    n_ch = M // (NW * CH)
    per = n_ch * CH
    mesh = plsc.VectorSubcoreMesh(
        core_axis_name="c", subcore_axis_name="s", num_cores=NC, num_subcores=NS)

    @functools.partial(
        pl.kernel,
        out_type=jax.ShapeDtypeStruct((M, DE), jnp.float32),
        mesh=mesh,
        compiler_params=pltpu.CompilerParams(use_tc_tiling_on_sc=False),
        scratch_types=[
            pltpu.VMEM((n_ch, CH), jnp.int32),
            pltpu.VMEM((2, CH, DE), jnp.float32),
            pltpu.SemaphoreType.DMA,
            pltpu.SemaphoreType.DMA,
        ],
    )
    def body(ei_hbm, ef_hbm, ef_out, ei_v, ebuf, se0, se1):
        wid = lax.axis_index("s") * NC + lax.axis_index("c")
        base = wid * per
        pltpu.sync_copy(ei_hbm.at[pl.ds(wid * n_ch, n_ch)], ei_v)
        se = (se0, se1)
        prev = None
        for j in range(n_ch):
            b = j % 2
            ce = pltpu.async_copy(ef_hbm.at[ei_v.at[j]], ebuf.at[b], se[b])
            if prev is not None:
                pce, pj, pb = prev
                pce.wait()
                pltpu.sync_copy(ebuf.at[pb], ef_out.at[pl.ds(base + pj * CH, CH)])
            prev = (ce, j, b)
        pce, pj, pb = prev
        pce.wait()
        pltpu.sync_copy(ebuf.at[pb], ef_out.at[pl.ds(base + pj * CH, CH)])

    return body(eidx2d, edge_feat)


def _tc_body(nn_ref, rs_ref, ts_ref, nts_ref, x_ref, nf_ref, ef_ref,
             tw_ref, tb_ref, wqx_ref, wqt_ref, wkn_ref, wke_ref, wkt_ref,
             wvn_ref, wve_ref, wvt_ref, w1o_ref, w1x_ref, b1_ref, w2_ref,
             b2_ref, out_ref):
    R = ts_ref.shape[0]
    f32 = jnp.float32
    dot = functools.partial(jnp.dot, preferred_element_type=f32)
    x = x_ref[...]
    nf = nf_ref[...]
    ef = ef_ref[...]
    tw = tw_ref[...]
    tb = tb_ref[...]
    delta = ts_ref[...] - nts_ref[...]
    te = _fast_cos(delta[:, :, None] * tw.reshape(1, 1, D) + tb.reshape(1, 1, D))
    te = te.reshape(R * K, D)
    kk = dot(nf, wkn_ref[...]) + dot(ef, wke_ref[...]) + dot(te, wkt_ref[...])
    vv = dot(nf, wvn_ref[...]) + dot(ef, wve_ref[...]) + dot(te, wvt_ref[...])
    q = dot(x, wqx_ref[...]) + dot(jnp.cos(tb), wqt_ref[...])
    k3 = kk.reshape(R, K, D)
    v3 = vv.reshape(R, K, D)
    nn = nn_ref[0, 0]
    kiota = lax.broadcasted_iota(jnp.int32, (R, K), 1)
    inv_sqrt = f32(1.0 / math.sqrt(DH))
    outs = []
    for h in range(H):
        sl = slice(h * DH, (h + 1) * DH)
        q_h = q[:, sl]
        k_h = k3[:, :, sl]
        v_h = v3[:, :, sl]
        scores = jnp.sum(q_h[:, None, :] * k_h, axis=-1) * inv_sqrt
        scores = jnp.where(kiota < nn, scores, -jnp.inf)
        m = jnp.max(scores, axis=1, keepdims=True)
        e = jnp.exp(scores - m)
        attn = e / jnp.sum(e, axis=1, keepdims=True)
        outs.append(jnp.sum(attn[:, :, None] * v_h, axis=1))
    out = jnp.concatenate(outs, axis=-1)
    hh = dot(out, w1o_ref[...]) + dot(x, w1x_ref[...]) + b1_ref[...]
    hh = jnp.maximum(hh, 0.0)
    out_ref[...] = dot(hh, w2_ref[...]) + b2_ref[...] + rs_ref[0, 0] * x


def _tc_embed(R, B3, interpret=False):
    G = B3 // R
    row = lambda i: (i, 0)
    fix = lambda i: (0, 0)
    smem = pl.BlockSpec(memory_space=pltpu.SMEM)
    return pl.pallas_call(
        _tc_body,
        grid=(G,),
        in_specs=[
            smem,
            smem,
            pl.BlockSpec((R, 1), row),
            pl.BlockSpec((R, K), row),
            pl.BlockSpec((R, D), row),
            pl.BlockSpec((R * K, D), row),
            pl.BlockSpec((R * K, DE), row),
            pl.BlockSpec((1, D), fix),
            pl.BlockSpec((1, D), fix),
            pl.BlockSpec((D, D), fix),
            pl.BlockSpec((D, D), fix),
            pl.BlockSpec((D, D), fix),
            pl.BlockSpec((DE, D), fix),
            pl.BlockSpec((D, D), fix),
            pl.BlockSpec((D, D), fix),
            pl.BlockSpec((DE, D), fix),
            pl.BlockSpec((D, D), fix),
            pl.BlockSpec((D, D), fix),
            pl.BlockSpec((D, D), fix),
            pl.BlockSpec((1, D), fix),
            pl.BlockSpec((D, D), fix),
            pl.BlockSpec((1, D), fix),
        ],
        out_specs=pl.BlockSpec((R, D), row),
        out_shape=jax.ShapeDtypeStruct((B3, D), jnp.float32),
        interpret=interpret,
    )


def kernel(source_nodes, destination_nodes, negative_nodes, edge_times, edge_idxs,
           n_neighbors, node_emb, edge_feat, nbr_nodes, nbr_eidx, nbr_times,
           time_w, time_b, Wq, Wk, Wv, W1, b1, W2, b2, res_scale):
    i32 = jnp.int32
    f32 = jnp.float32
    node_emb = node_emb.astype(f32)
    edge_feat = edge_feat.astype(f32)
    ts = edge_times.astype(f32)

    Nn = nbr_nodes.shape[0]
    tbl = jnp.concatenate([
        nbr_nodes.astype(i32),
        nbr_eidx.astype(i32),
        lax.bitcast_convert_type(nbr_times.astype(f32), i32),
        jnp.zeros((Nn, TBLW - 3 * K), i32),
    ], axis=1)

    NC, NS = _sc_dims()
    NW = NC * NS
    nn = jnp.asarray(n_neighbors, i32).reshape(1, 1)
    rs = jnp.asarray(res_scale, f32).reshape(1, 1)
    tw2 = time_w.astype(f32).reshape(1, D)
    tb2 = time_b.astype(f32).reshape(1, D)
    R = 256

    outs = []
    for grp in (source_nodes, destination_nodes, negative_nodes):
        nodes = grp.astype(i32)
        Bs = nodes.shape[0]
        Ms = Bs * K
        g = _sc_gather_tables(nodes, tbl)
        nbrs3d = g[:, :K].reshape(NW, Ms // (NW * 128), 128)
        eidx2d = g[:, K:2 * K].reshape(Ms // 128, 128)
        nts = lax.bitcast_convert_type(g[:, 2 * K:3 * K], f32)
        x, nf = _sc_gather_embeddings(nodes, nbrs3d, node_emb, Bs, Ms)
        ef = _sc_gather_edges(eidx2d, edge_feat, Ms)
        emb = _tc_embed(R, Bs)(
            nn, rs, ts.reshape(Bs, 1), nts, x, nf, ef, tw2, tb2,
            Wq[:D], Wq[D:], Wk[:D], Wk[D:D + DE], Wk[D + DE:],
            Wv[:D], Wv[D:D + DE], Wv[D + DE:],
            W1[:D], W1[D:], b1.reshape(1, D), W2, b2.reshape(1, D))
        outs.append(emb)
    return tuple(outs)
